# baseline (speedup 1.0000x reference)
import functools

import jax
import jax.numpy as jnp
from jax import lax
from jax.experimental import pallas as pl
from jax.experimental.pallas import tpu as pltpu
from jax.experimental.pallas import tpu_sc as plsc

VOCAB_N = 100000
NCORES = 2
NSUB = 16
LANES = 16
HALF = VOCAB_N // NCORES
BMAP = 1792
MSLICE = BMAP // NSUB
HPAD = 32 * BMAP
N_PRED_A = 16384 * 48
N_PRED_B = 16384 * 2
N_CAPT = 16384 * 200
PRED_A_PER_TILE = N_PRED_A // NSUB
PRED_B_PER_TILE = N_PRED_B // NSUB
CAPT_PER_TILE = N_CAPT // NSUB
CHUNK = 25600

_mesh = plsc.VectorSubcoreMesh(core_axis_name="c", subcore_axis_name="s")


@functools.partial(
    pl.kernel,
    out_type=jax.ShapeDtypeStruct((NCORES * NSUB, 2, LANES), jnp.int32),
    mesh=_mesh,
    scratch_types=[
        pltpu.VMEM((HPAD,), jnp.int32),
        pltpu.VMEM((2, CHUNK), jnp.int32),
        pltpu.VMEM((BMAP,), jnp.int32),
        pltpu.VMEM((BMAP,), jnp.int32),
        pltpu.VMEM((2, LANES), jnp.int32),
        pltpu.VMEM_SHARED((NSUB, BMAP), jnp.int32),
        pltpu.SemaphoreType.DMA,
        pltpu.SemaphoreType.DMA,
        pltpu.SemaphoreType.DMA,
    ],
    compiler_params=pltpu.CompilerParams(use_tc_tiling_on_sc=False,
                                         needs_layout_passes=False),
)
def _vocab_usage_sc(preds_a_hbm, preds_b_hbm, capts_hbm, out_hbm,
                    pres, tbuf, pkbuf, mbuf, cbuf, shared,
                    sem_a, sem_b, sem_m):
    core = lax.axis_index("c")
    sub = lax.axis_index("s")
    wid = core * NSUB + sub
    base = core * HALF
    zeros16 = jnp.zeros((LANES,), jnp.int32)
    ones16 = jnp.ones((LANES,), jnp.int32)
    sems = (sem_a, sem_b)

    def _zero_pres():
        @plsc.parallel_loop(0, HPAD // LANES, unroll=8)
        def _z(i):
            pres[pl.ds(i * LANES, LANES)] = zeros16

    def _chunks(per_tile):
        out, st = [], 0
        while st < per_tile:
            sz = min(CHUNK, per_tile - st)
            out.append((st, sz))
            st += sz
        return out

    def _issue(src_hbm, per_tile, st, sz, b):
        pltpu.async_copy(src_hbm.at[pl.ds(sub * per_tile + st, sz)],
                         tbuf.at[b, pl.ds(0, sz)], sems[b])

    def _prefetch(items, depth=1):
        for idx in range(min(depth, len(items))):
            src_hbm, per_tile, (st, sz) = items[idx]
            _issue(src_hbm, per_tile, st, sz, idx % 2)

    def _scatter(items, preissued=1):
        for idx, (src_hbm, per_tile, (st, sz)) in enumerate(items):
            b = idx % 2
            if idx + 1 < len(items) and idx + 1 >= preissued:
                s2, p2, (st2, sz2) = items[idx + 1]
                _issue(s2, p2, st2, sz2, (idx + 1) % 2)
            pltpu.make_async_copy(src_hbm.at[pl.ds(sub * per_tile, sz)],
                                  tbuf.at[b, pl.ds(0, sz)], sems[b]).wait()

            @plsc.parallel_loop(0, sz // LANES, unroll=16)
            def _v(i):
                tok = tbuf[b, pl.ds(i * LANES, LANES)]
                loc = tok - base
                msk = loc.astype(jnp.uint32) < jnp.uint32(HALF)
                plsc.store_scatter(pres, [loc], ones16, mask=msk)

    def _pack(clear):
        @plsc.parallel_loop(0, BMAP // LANES, unroll=2)
        def _p(i):
            acc = pres[pl.ds(i * LANES, LANES)]
            if clear:
                pres[pl.ds(i * LANES, LANES)] = zeros16
            for j in range(1, 32):
                v = pres[pl.ds(j * BMAP + i * LANES, LANES)]
                if clear:
                    pres[pl.ds(j * BMAP + i * LANES, LANES)] = zeros16
                acc = acc | (v << j)
            pkbuf[pl.ds(i * LANES, LANES)] = acc

    def _merge_count(inp):
        descs = [pltpu.async_copy(shared.at[t, pl.ds(sub * MSLICE, MSLICE)],
                                  mbuf.at[pl.ds(t * MSLICE, MSLICE)], sem_m)
                 for t in range(NSUB)]
        for d in descs:
            d.wait()

        @plsc.parallel_loop(0, MSLICE // LANES, unroll=1, carry=zeros16)
        def _cnt(j, cv):
            acc = mbuf[pl.ds(j * LANES, LANES)]
            for t in range(1, NSUB):
                acc = acc | mbuf[pl.ds(t * MSLICE + j * LANES, LANES)]
            u = plsc.bitcast(acc, jnp.uint32)
            u = u - ((u >> jnp.uint32(1)) & jnp.uint32(0x55555555))
            u = ((u & jnp.uint32(0x33333333))
                 + ((u >> jnp.uint32(2)) & jnp.uint32(0x33333333)))
            u = (u + (u >> jnp.uint32(4))) & jnp.uint32(0x0F0F0F0F)
            u = (u * jnp.uint32(0x01010101)) >> jnp.uint32(24)
            return cv + plsc.bitcast(u, jnp.int32)
        cbuf[inp] = _cnt

    pred_items = ([(preds_a_hbm, PRED_A_PER_TILE, c)
                   for c in _chunks(PRED_A_PER_TILE)]
                  + [(preds_b_hbm, PRED_B_PER_TILE, c)
                     for c in _chunks(PRED_B_PER_TILE)])
    capt_items = [(capts_hbm, CAPT_PER_TILE, c)
                  for c in _chunks(CAPT_PER_TILE)]

    _prefetch(pred_items)
    _zero_pres()
    _scatter(pred_items)
    _prefetch(capt_items, depth=2)
    _pack(clear=True)
    pltpu.sync_copy(pkbuf, shared.at[sub])
    plsc.subcore_barrier()
    _merge_count(0)
    plsc.subcore_barrier()
    _scatter(capt_items, preissued=2)
    _pack(clear=False)
    pltpu.sync_copy(pkbuf, shared.at[sub])
    plsc.subcore_barrier()
    _merge_count(1)

    pltpu.sync_copy(cbuf, out_hbm.at[wid])


def kernel(preds, captions):
    pa = preds[:, :48].reshape(128, 128, 6, 8).transpose(2, 0, 3, 1).reshape(-1)
    pb = preds[:, 48:].T.reshape(-1)
    cf = captions.reshape(128, 128, 25, 8).transpose(2, 0, 3, 1).reshape(-1)
    parts = _vocab_usage_sc(pa, pb, cf)
    n_pred = parts[:, 0, :].sum().astype(jnp.float32)
    n_capt = parts[:, 1, :].sum().astype(jnp.float32)
    return jnp.where(n_capt > 0, n_pred / jnp.maximum(n_capt, 1.0),
                     jnp.float32(0.0))


# speedup vs baseline: 1.0158x; 1.0158x over previous
import functools

import jax
import jax.numpy as jnp
from jax import lax
from jax.experimental import pallas as pl
from jax.experimental.pallas import tpu as pltpu
from jax.experimental.pallas import tpu_sc as plsc

VOCAB_N = 100000
NCORES = 2
NSUB = 16
LANES = 16
HALF = VOCAB_N // NCORES
BMAP = 1792
MSLICE = BMAP // NSUB
HPAD = 32 * BMAP
N_PRED_A = 16384 * 48
N_PRED_B = 16384 * 2
N_CAPT = 16384 * 200
PRED_A_PER_TILE = N_PRED_A // NSUB
PRED_B_PER_TILE = N_PRED_B // NSUB
CAPT_PER_TILE = N_CAPT // NSUB
CHUNK = 25600

_mesh = plsc.VectorSubcoreMesh(core_axis_name="c", subcore_axis_name="s")


@functools.partial(
    pl.kernel,
    out_type=jax.ShapeDtypeStruct((NCORES * NSUB, 2, LANES), jnp.int32),
    mesh=_mesh,
    scratch_types=[
        pltpu.VMEM((HPAD,), jnp.int32),
        pltpu.VMEM((2, CHUNK), jnp.int32),
        pltpu.VMEM((BMAP,), jnp.int32),
        pltpu.VMEM((BMAP,), jnp.int32),
        pltpu.VMEM((2, LANES), jnp.int32),
        pltpu.VMEM_SHARED((NSUB, BMAP), jnp.int32),
        pltpu.SemaphoreType.DMA,
        pltpu.SemaphoreType.DMA,
        pltpu.SemaphoreType.DMA,
    ],
    compiler_params=pltpu.CompilerParams(use_tc_tiling_on_sc=False,
                                         needs_layout_passes=False),
)
def _vocab_usage_sc(preds_a_hbm, preds_b_hbm, capts_hbm, out_hbm,
                    pres, tbuf, pkbuf, mbuf, cbuf, shared,
                    sem_a, sem_b, sem_m):
    core = lax.axis_index("c")
    sub = lax.axis_index("s")
    wid = core * NSUB + sub
    base = core * HALF
    zeros16 = jnp.zeros((LANES,), jnp.int32)
    ones16 = jnp.ones((LANES,), jnp.int32)
    sems = (sem_a, sem_b)

    def _zero_pres():
        @plsc.parallel_loop(0, HPAD // LANES, unroll=8)
        def _z(i):
            pres[pl.ds(i * LANES, LANES)] = zeros16

    def _chunks(per_tile):
        out, st = [], 0
        while st < per_tile:
            sz = min(CHUNK, per_tile - st)
            out.append((st, sz))
            st += sz
        return out

    def _issue(src_hbm, per_tile, st, sz, b):
        pltpu.async_copy(src_hbm.at[pl.ds(sub * per_tile + st, sz)],
                         tbuf.at[b, pl.ds(0, sz)], sems[b])

    def _prefetch(items, depth=1):
        for idx in range(min(depth, len(items))):
            src_hbm, per_tile, (st, sz) = items[idx]
            _issue(src_hbm, per_tile, st, sz, idx % 2)

    def _scatter(items, preissued=1):
        for idx, (src_hbm, per_tile, (st, sz)) in enumerate(items):
            b = idx % 2
            if idx + 1 < len(items) and idx + 1 >= preissued:
                s2, p2, (st2, sz2) = items[idx + 1]
                _issue(s2, p2, st2, sz2, (idx + 1) % 2)
            pltpu.make_async_copy(src_hbm.at[pl.ds(sub * per_tile, sz)],
                                  tbuf.at[b, pl.ds(0, sz)], sems[b]).wait()

            @plsc.parallel_loop(0, sz // LANES, unroll=16)
            def _v(i):
                tok = tbuf[b, pl.ds(i * LANES, LANES)]
                loc = tok - base
                msk = loc.astype(jnp.uint32) < jnp.uint32(HALF)
                plsc.store_scatter(pres, [loc], ones16, mask=msk)

    def _pack(clear):
        @plsc.parallel_loop(0, BMAP // LANES, unroll=2)
        def _p(i):
            acc = pres[pl.ds(i * LANES, LANES)]
            if clear:
                pres[pl.ds(i * LANES, LANES)] = zeros16
            for j in range(1, 32):
                v = pres[pl.ds(j * BMAP + i * LANES, LANES)]
                if clear:
                    pres[pl.ds(j * BMAP + i * LANES, LANES)] = zeros16
                acc = acc | (v << j)
            pkbuf[pl.ds(i * LANES, LANES)] = acc

    def _merge_count(inp):
        descs = [pltpu.async_copy(shared.at[t, pl.ds(sub * MSLICE, MSLICE)],
                                  mbuf.at[pl.ds(t * MSLICE, MSLICE)], sem_m)
                 for t in range(NSUB)]
        for d in descs:
            d.wait()

        @plsc.parallel_loop(0, MSLICE // LANES, unroll=1, carry=zeros16)
        def _cnt(j, cv):
            acc = mbuf[pl.ds(j * LANES, LANES)]
            for t in range(1, NSUB):
                acc = acc | mbuf[pl.ds(t * MSLICE + j * LANES, LANES)]
            u = plsc.bitcast(acc, jnp.uint32)
            u = u - ((u >> jnp.uint32(1)) & jnp.uint32(0x55555555))
            u = ((u & jnp.uint32(0x33333333))
                 + ((u >> jnp.uint32(2)) & jnp.uint32(0x33333333)))
            u = (u + (u >> jnp.uint32(4))) & jnp.uint32(0x0F0F0F0F)
            u = (u * jnp.uint32(0x01010101)) >> jnp.uint32(24)
            return cv + plsc.bitcast(u, jnp.int32)
        cbuf[inp] = _cnt

    pred_items = ([(preds_a_hbm, PRED_A_PER_TILE, c)
                   for c in _chunks(PRED_A_PER_TILE)]
                  + [(preds_b_hbm, PRED_B_PER_TILE, c)
                     for c in _chunks(PRED_B_PER_TILE)])
    capt_items = [(capts_hbm, CAPT_PER_TILE, c)
                  for c in _chunks(CAPT_PER_TILE)]

    _prefetch(pred_items)
    _zero_pres()
    _scatter(pred_items)
    _prefetch(capt_items)
    _pack(clear=True)
    pltpu.sync_copy(pkbuf, shared.at[sub])
    plsc.subcore_barrier()
    _merge_count(0)
    plsc.subcore_barrier()
    _scatter(capt_items)
    _pack(clear=False)
    pltpu.sync_copy(pkbuf, shared.at[sub])
    plsc.subcore_barrier()
    _merge_count(1)

    pltpu.sync_copy(cbuf, out_hbm.at[wid])


def kernel(preds, captions):
    pa = preds[:, :48].reshape(128, 128, 6, 8).transpose(2, 0, 3, 1).reshape(-1)
    pb = preds[:, 48:].T.reshape(-1)
    cf = captions.reshape(128, 128, 25, 8).transpose(2, 0, 3, 1).reshape(-1)
    parts = _vocab_usage_sc(pa, pb, cf)
    n_pred = parts[:, 0, :].sum().astype(jnp.float32)
    n_capt = parts[:, 1, :].sum().astype(jnp.float32)
    return jnp.where(n_capt > 0, n_pred / jnp.maximum(n_capt, 1.0),
                     jnp.float32(0.0))
